# Initial kernel scaffold; baseline (speedup 1.0000x reference)
import jax
import jax.numpy as jnp
from jax.experimental import pallas as pl


def kernel(x, table):
    raise NotImplementedError("write your pallas kernel here")


# speedup vs baseline: 1.7939x; 1.7939x over previous
import functools

import jax
import jax.numpy as jnp
from jax import lax
from jax.experimental import pallas as pl
from jax.experimental.pallas import tpu as pltpu
from jax.experimental.pallas import tpu_sc as plsc

EMBED = 64
NUM_CORES = 2
NUM_SUBCORES = 16
NW = NUM_CORES * NUM_SUBCORES
CHUNK = 512


def _sc_gather(x_flat, table):
    n = x_flat.shape[0]
    per_w = n // NW
    steps = per_w // CHUNK
    assert per_w % CHUNK == 0 and n % NW == 0

    mesh = plsc.VectorSubcoreMesh(core_axis_name="c", subcore_axis_name="s")

    @functools.partial(
        pl.kernel,
        mesh=mesh,
        out_type=jax.ShapeDtypeStruct((n, EMBED), jnp.float32),
        scratch_types=[
            pltpu.VMEM((CHUNK,), jnp.int32),
            pltpu.VMEM((CHUNK, EMBED), jnp.float32),
            pltpu.SemaphoreType.DMA,
        ],
        compiler_params=pltpu.CompilerParams(use_tc_tiling_on_sc=False),
    )
    def k(idx_hbm, table_hbm, out_hbm, idx_v, rows_v, sem):
        wid = lax.axis_index("s") * NUM_CORES + lax.axis_index("c")
        base = wid * per_w

        def body(i, carry):
            off = base + i * CHUNK
            pltpu.sync_copy(idx_hbm.at[pl.ds(off, CHUNK)], idx_v)
            pltpu.async_copy(table_hbm.at[idx_v], rows_v, sem).wait()
            pltpu.sync_copy(rows_v, out_hbm.at[pl.ds(off, CHUNK)])
            return carry

        lax.fori_loop(0, steps, body, 0)

    return k(x_flat, table)


def kernel(x, table):
    b, l = x.shape
    x_flat = x.reshape(b * l).astype(jnp.int32)
    out = _sc_gather(x_flat, table)
    return out.reshape(b, l, EMBED)


# speedup vs baseline: 1.8755x; 1.0455x over previous
import functools

import jax
import jax.numpy as jnp
from jax import lax
from jax.experimental import pallas as pl
from jax.experimental.pallas import tpu as pltpu
from jax.experimental.pallas import tpu_sc as plsc

EMBED = 64
NUM_CORES = 2
NUM_SUBCORES = 16
NW = NUM_CORES * NUM_SUBCORES
CHUNK = 512
NBUF = 2


def _sc_gather(x_flat, table):
    n = x_flat.shape[0]
    per_w = n // NW
    steps = per_w // CHUNK
    assert n % NW == 0 and per_w % CHUNK == 0 and steps % NBUF == 0

    mesh = plsc.VectorSubcoreMesh(core_axis_name="c", subcore_axis_name="s")

    @functools.partial(
        pl.kernel,
        mesh=mesh,
        out_type=jax.ShapeDtypeStruct((n, EMBED), jnp.float32),
        scratch_types=[
            pltpu.VMEM((per_w,), jnp.int32),
            pltpu.VMEM((NBUF, CHUNK, EMBED), jnp.float32),
        ]
        + [pltpu.SemaphoreType.DMA] * (2 * NBUF),
        compiler_params=pltpu.CompilerParams(use_tc_tiling_on_sc=False),
    )
    def k(idx_hbm, table_hbm, out_hbm, idx_all, rows, *sems):
        gsems = sems[:NBUF]
        ssems = sems[NBUF:]
        wid = lax.axis_index("s") * NUM_CORES + lax.axis_index("c")
        base = wid * per_w
        pltpu.sync_copy(idx_hbm.at[pl.ds(base, per_w)], idx_all)

        def g_start(c, b):
            pltpu.async_copy(
                table_hbm.at[idx_all.at[pl.ds(c * CHUNK, CHUNK)]],
                rows.at[b], gsems[b])

        def g_wait(b):
            pltpu.make_async_copy(
                table_hbm.at[idx_all.at[pl.ds(0, CHUNK)]],
                rows.at[b], gsems[b]).wait()

        def s_start(c, b):
            pltpu.async_copy(
                rows.at[b],
                out_hbm.at[pl.ds(base + c * CHUNK, CHUNK)], ssems[b])

        def s_wait(b):
            pltpu.make_async_copy(
                rows.at[b],
                out_hbm.at[pl.ds(base, CHUNK)], ssems[b]).wait()

        for b in range(NBUF):
            g_start(b, b)

        def body(g, carry):
            for b in range(NBUF):
                c = g * NBUF + b
                g_wait(b)
                s_start(c, b)
                s_wait(b)
                g_start(c + NBUF, b)
            return carry

        lax.fori_loop(0, steps // NBUF - 1, body, 0)

        c_last = steps - NBUF
        for b in range(NBUF):
            g_wait(b)
            s_start(c_last + b, b)
        for b in range(NBUF):
            s_wait(b)

    return k(x_flat, table)


def kernel(x, table):
    b, l = x.shape
    x_flat = x.reshape(b * l).astype(jnp.int32)
    out = _sc_gather(x_flat, table)
    return out.reshape(b, l, EMBED)
